# baseline (speedup 1.0000x reference)
import functools

import jax
import jax.numpy as jnp
from jax import lax
from jax.experimental import pallas as pl
from jax.experimental.pallas import tpu as pltpu
from jax.experimental.pallas import tpu_sc as plsc

_VOCAB = 1000000
_D = 64
_C = 2
_B = 4096
_H = 200

_NC = 2
_NS = 16
_NW = _NC * _NS
_SPW = _B // _NW
_CHUNK0 = 128
_CHUNK1 = _H - _CHUNK0
_NREG = _D // 16


def _sc_body(x_ref, tab_ref, out_ref, idx_v, rows_v, sums_v, sem):
    wid = lax.axis_index("s") * _NC + lax.axis_index("c")
    base = pl.multiple_of(wid * _SPW, _SPW)
    pltpu.sync_copy(x_ref.at[pl.ds(base * _H, _SPW * _H)], idx_v)

    def sample_body(s, carry):
        off = pl.multiple_of(s * _H, 8)
        cp0 = pltpu.async_copy(
            tab_ref.at[idx_v.at[pl.ds(off, _CHUNK0)]],
            rows_v.at[pl.ds(0, _CHUNK0)], sem)
        cp1 = pltpu.async_copy(
            tab_ref.at[idx_v.at[pl.ds(off + _CHUNK0, _CHUNK1)]],
            rows_v.at[pl.ds(_CHUNK0, _CHUNK1)], sem)
        cp0.wait()
        cp1.wait()

        def acc_body(t, accs):
            return tuple(accs[j] + rows_v[t, pl.ds(j * 16, 16)]
                         for j in range(_NREG))

        accs = lax.fori_loop(
            0, _H, acc_body,
            tuple(jnp.zeros((16,), jnp.float32) for _ in range(_NREG)))
        for j in range(_NREG):
            sums_v[s, pl.ds(j * 16, 16)] = accs[j]
        return carry

    lax.fori_loop(0, _SPW, sample_body, 0)
    pltpu.sync_copy(sums_v, out_ref.at[pl.ds(base, _SPW)])


def _sc_pool(x_flat, emb_table):
    mesh = plsc.VectorSubcoreMesh(core_axis_name="c", subcore_axis_name="s")
    f = pl.kernel(
        _sc_body,
        out_type=jax.ShapeDtypeStruct((_B, _D), jnp.float32),
        mesh=mesh,
        scratch_types=[
            pltpu.VMEM((_SPW * _H,), jnp.int32),
            pltpu.VMEM((_H, _D), jnp.float32),
            pltpu.VMEM((_SPW, _D), jnp.float32),
            pltpu.SemaphoreType.DMA,
        ],
        compiler_params=pltpu.CompilerParams(use_tc_tiling_on_sc=False),
    )
    return f(x_flat, emb_table)


def _fc_body(sums_ref, w_ref, b_ref, out_ref):
    pooled = sums_ref[...] * (1.0 / _H)
    out_ref[...] = lax.dot_general(
        pooled, w_ref[...], (((1,), (1,)), ((), ())),
        preferred_element_type=jnp.float32) + b_ref[...]


def _fc(sums, fc_w, fc_b2d):
    return pl.pallas_call(
        _fc_body,
        out_shape=jax.ShapeDtypeStruct((_B, _C), jnp.float32),
    )(sums, fc_w, fc_b2d)


def kernel(x, emb_table, fc_w, fc_b):
    sums = _sc_pool(x.reshape(-1), emb_table)
    return _fc(sums, fc_w, fc_b.reshape(1, _C))


# speedup vs baseline: 1.2375x; 1.2375x over previous
import functools

import jax
import jax.numpy as jnp
from jax import lax
from jax.experimental import pallas as pl
from jax.experimental.pallas import tpu as pltpu
from jax.experimental.pallas import tpu_sc as plsc

_VOCAB = 1000000
_D = 64
_C = 2
_B = 4096
_H = 200

_NC = 2
_NS = 16
_NW = _NC * _NS
_SPW = _B // _NW
_CHUNK0 = 128
_CHUNK1 = _H - _CHUNK0
_NREG = _D // 16


_NBUF = 4


def _sc_body(x_ref, tab_ref, out_ref, idx_v, sums_v, bufs, sems):
    wid = lax.axis_index("s") * _NC + lax.axis_index("c")
    base = pl.multiple_of(wid * _SPW, _SPW)
    pltpu.sync_copy(x_ref.at[pl.ds(base * _H, _SPW * _H)], idx_v)

    def fire(s, buf, sem):
        off = pl.multiple_of(s * _H, 8)
        pltpu.async_copy(
            tab_ref.at[idx_v.at[pl.ds(off, _CHUNK0)]],
            buf.at[pl.ds(0, _CHUNK0)], sem)
        pltpu.async_copy(
            tab_ref.at[idx_v.at[pl.ds(off + _CHUNK0, _CHUNK1)]],
            buf.at[pl.ds(_CHUNK0, _CHUNK1)], sem)

    def drain(buf, sem):
        pltpu.make_async_copy(
            tab_ref.at[pl.ds(0, _CHUNK0)], buf.at[pl.ds(0, _CHUNK0)],
            sem).wait()
        pltpu.make_async_copy(
            tab_ref.at[pl.ds(0, _CHUNK1)], buf.at[pl.ds(_CHUNK0, _CHUNK1)],
            sem).wait()

    for b in range(_NBUF):
        fire(b, bufs[b], sems[b])

    def group_body(g, carry):
        s0 = g * _NBUF
        for b in range(_NBUF):
            s = s0 + b
            buf, sem = bufs[b], sems[b]
            drain(buf, sem)

            def acc_body(t, accs):
                return tuple(accs[j] + buf[t, pl.ds(j * 16, 16)]
                             for j in range(_NREG))

            accs = lax.fori_loop(
                0, _H, acc_body,
                tuple(jnp.zeros((16,), jnp.float32) for _ in range(_NREG)),
                unroll=8)
            for j in range(_NREG):
                sums_v[s, pl.ds(j * 16, 16)] = accs[j]

            @pl.when(s + _NBUF < _SPW)
            def _():
                fire(s + _NBUF, buf, sem)
        return carry

    lax.fori_loop(0, _SPW // _NBUF, group_body, 0)
    pltpu.sync_copy(sums_v, out_ref.at[pl.ds(base, _SPW)])


def _sc_pool(x_flat, emb_table):
    mesh = plsc.VectorSubcoreMesh(core_axis_name="c", subcore_axis_name="s")

    def body(x_ref, tab_ref, out_ref, idx_v, sums_v, *rest):
        bufs = rest[:_NBUF]
        sems = rest[_NBUF:]
        _sc_body(x_ref, tab_ref, out_ref, idx_v, sums_v, bufs, sems)

    f = pl.kernel(
        body,
        out_type=jax.ShapeDtypeStruct((_B, _D), jnp.float32),
        mesh=mesh,
        scratch_types=(
            [pltpu.VMEM((_SPW * _H,), jnp.int32),
             pltpu.VMEM((_SPW, _D), jnp.float32)]
            + [pltpu.VMEM((_H, _D), jnp.float32) for _ in range(_NBUF)]
            + [pltpu.SemaphoreType.DMA for _ in range(_NBUF)]
        ),
        compiler_params=pltpu.CompilerParams(use_tc_tiling_on_sc=False),
    )
    return f(x_flat, emb_table)


def _fc_body(sums_ref, w_ref, b_ref, out_ref):
    pooled = sums_ref[...] * (1.0 / _H)
    out_ref[...] = lax.dot_general(
        pooled, w_ref[...], (((1,), (1,)), ((), ())),
        preferred_element_type=jnp.float32) + b_ref[...]


def _fc(sums, fc_w, fc_b2d):
    return pl.pallas_call(
        _fc_body,
        out_shape=jax.ShapeDtypeStruct((_B, _C), jnp.float32),
    )(sums, fc_w, fc_b2d)


def kernel(x, emb_table, fc_w, fc_b):
    sums = _sc_pool(x.reshape(-1), emb_table)
    return _fc(sums, fc_w, fc_b.reshape(1, _C))


# speedup vs baseline: 1.8752x; 1.5153x over previous
import jax
import jax.numpy as jnp
from jax import lax
from jax.experimental import pallas as pl
from jax.experimental.pallas import tpu as pltpu
from jax.experimental.pallas import tpu_sc as plsc

_VOCAB = 1000000
_D = 64
_C = 2
_B = 4096
_H = 200

_NC = 2
_NS = 16
_NW = _NC * _NS
_SPW = _B // _NW
_IPW = _SPW * _H
_NCHUNK = _IPW // 128
_GPAD = 16
_NV = _H // 16
_TAILR = _H - _NV * 16


_W1 = 2048


def _mm_body(w_ref, tabt_ref, out_ref):
    w = w_ref[...] * (1.0 / _H)
    out_ref[...] = lax.dot_general(
        w, tabt_ref[...], (((1,), (0,)), ((), ())),
        preferred_element_type=jnp.float32)


def _mm(fc_w, tabt):
    grid = (pl.cdiv(_VOCAB, _W1),)
    return pl.pallas_call(
        _mm_body,
        grid=grid,
        in_specs=[
            pl.BlockSpec((_C, _D), lambda i: (0, 0)),
            pl.BlockSpec((_D, _W1), lambda i: (0, i)),
        ],
        out_specs=pl.BlockSpec((_C, _W1), lambda i: (0, i)),
        out_shape=jax.ShapeDtypeStruct((_C, _VOCAB), jnp.float32),
    )(fc_w, tabt)


def _sc_body(x_ref, tw0_ref, tw1_ref, out_ref, idx_v, g0, g1, outb, sem):
    wid = lax.axis_index("s") * _NC + lax.axis_index("c")
    base = pl.multiple_of(wid * _SPW, _SPW)
    pltpu.sync_copy(x_ref.at[pl.ds(base * _H, _IPW)], idx_v)

    def fire(c, carry):
        off = pl.multiple_of(c * 128, 128)
        ids = idx_v.at[pl.ds(off, 128)]
        pltpu.async_copy(tw0_ref.at[ids], g0.at[pl.ds(off, 128)], sem)
        pltpu.async_copy(tw1_ref.at[ids], g1.at[pl.ds(off, 128)], sem)
        return carry

    lax.fori_loop(0, _NCHUNK, fire, 0)
    pltpu.make_async_copy(
        tw0_ref.at[pl.ds(0, _IPW)], g0.at[pl.ds(0, _IPW)], sem).wait()
    pltpu.make_async_copy(
        tw1_ref.at[pl.ds(0, _IPW)], g1.at[pl.ds(0, _IPW)], sem).wait()

    lanes = lax.broadcasted_iota(jnp.int32, (16,), 0)
    tail_mask = lanes < _TAILR

    def acc_sample(s, carry):
        off = pl.multiple_of(s * _H, 8)
        acc0 = jnp.zeros((16,), jnp.float32)
        acc1 = jnp.zeros((16,), jnp.float32)
        for i in range(_NV):
            acc0 = acc0 + g0[pl.ds(off + i * 16, 16)]
            acc1 = acc1 + g1[pl.ds(off + i * 16, 16)]
        t0 = g0[pl.ds(off + _NV * 16, 16)]
        t1 = g1[pl.ds(off + _NV * 16, 16)]
        acc0 = acc0 + jnp.where(tail_mask, t0, 0.0)
        acc1 = acc1 + jnp.where(tail_mask, t1, 0.0)
        outb[s, pl.ds(0, 16)] = acc0
        outb[s, pl.ds(16, 16)] = acc1
        return carry

    lax.fori_loop(0, _SPW, acc_sample, 0)
    pltpu.sync_copy(outb, out_ref.at[pl.ds(base, _SPW)])


def _sc_gather(x_flat, tw0, tw1):
    mesh = plsc.VectorSubcoreMesh(core_axis_name="c", subcore_axis_name="s")
    f = pl.kernel(
        _sc_body,
        out_type=jax.ShapeDtypeStruct((_B, 32), jnp.float32),
        mesh=mesh,
        scratch_types=[
            pltpu.VMEM((_IPW,), jnp.int32),
            pltpu.VMEM((_IPW + _GPAD,), jnp.float32),
            pltpu.VMEM((_IPW + _GPAD,), jnp.float32),
            pltpu.VMEM((_SPW, 32), jnp.float32),
            pltpu.SemaphoreType.DMA,
        ],
        compiler_params=pltpu.CompilerParams(use_tc_tiling_on_sc=False),
    )
    return f(x_flat, tw0, tw1)


def _fold_body(part_ref, b_ref, out_ref):
    j = lax.broadcasted_iota(jnp.int32, (32, _C), 0)
    k = lax.broadcasted_iota(jnp.int32, (32, _C), 1)
    sel = ((j // 16) == k).astype(jnp.float32)
    out_ref[...] = lax.dot_general(
        part_ref[...], sel, (((1,), (0,)), ((), ())),
        preferred_element_type=jnp.float32) + b_ref[...]


def _fold(partial, fc_b2d):
    return pl.pallas_call(
        _fold_body,
        out_shape=jax.ShapeDtypeStruct((_B, _C), jnp.float32),
    )(partial, fc_b2d)


def kernel(x, emb_table, fc_w, fc_b):
    twt = _mm(fc_w, emb_table.T)
    partial = _sc_gather(x.reshape(-1), twt[0], twt[1])
    return _fold(partial, fc_b.reshape(1, _C))


# speedup vs baseline: 3.8349x; 2.0450x over previous
import jax
import jax.numpy as jnp
from jax import lax
from jax.experimental import pallas as pl
from jax.experimental.pallas import tpu as pltpu
from jax.experimental.pallas import tpu_sc as plsc

_VOCAB = 1000000
_D = 64
_C = 2
_B = 4096
_H = 200

_NC = 2
_NS = 16
_NW = _NC * _NS
_SPW = _B // _NW
_IPW = _SPW * _H
_NCHUNK = _IPW // 128
_GPAD = 16
_NV = _H // 16
_TAILR = _H - _NV * 16


_W1 = 8192


def _mm_body(w_ref, tabt_ref, out0_ref, out1_ref):
    w = w_ref[...] * (1.0 / _H)
    res = lax.dot_general(
        w, tabt_ref[...], (((1,), (0,)), ((), ())),
        preferred_element_type=jnp.float32)
    out0_ref[...] = res[0, :]
    out1_ref[...] = res[1, :]


def _mm(fc_w, tabt):
    grid = (pl.cdiv(_VOCAB, _W1),)
    return pl.pallas_call(
        _mm_body,
        grid=grid,
        in_specs=[
            pl.BlockSpec((_C, _D), lambda i: (0, 0)),
            pl.BlockSpec((_D, _W1), lambda i: (0, i)),
        ],
        out_specs=[
            pl.BlockSpec((_W1,), lambda i: (i,)),
            pl.BlockSpec((_W1,), lambda i: (i,)),
        ],
        out_shape=[
            jax.ShapeDtypeStruct((_VOCAB,), jnp.float32),
            jax.ShapeDtypeStruct((_VOCAB,), jnp.float32),
        ],
    )(fc_w, tabt)


def _sc_body(x_ref, tw0_ref, tw1_ref, out_ref, idx_v, g0, g1, outb, sem):
    wid = lax.axis_index("s") * _NC + lax.axis_index("c")
    base = pl.multiple_of(wid * _SPW, _SPW)
    pltpu.sync_copy(x_ref.at[pl.ds(base * _H, _IPW)], idx_v)

    def fire(c, carry):
        off = pl.multiple_of(c * 128, 128)
        ids = idx_v.at[pl.ds(off, 128)]
        pltpu.async_copy(tw0_ref.at[ids], g0.at[pl.ds(off, 128)], sem)
        pltpu.async_copy(tw1_ref.at[ids], g1.at[pl.ds(off, 128)], sem)
        return carry

    lax.fori_loop(0, _NCHUNK, fire, 0)
    pltpu.make_async_copy(
        tw0_ref.at[pl.ds(0, _IPW)], g0.at[pl.ds(0, _IPW)], sem).wait()
    pltpu.make_async_copy(
        tw1_ref.at[pl.ds(0, _IPW)], g1.at[pl.ds(0, _IPW)], sem).wait()

    lanes = lax.broadcasted_iota(jnp.int32, (16,), 0)
    tail_mask = lanes < _TAILR

    def acc_sample(s, carry):
        off = pl.multiple_of(s * _H, 8)
        acc0 = jnp.zeros((16,), jnp.float32)
        acc1 = jnp.zeros((16,), jnp.float32)
        for i in range(_NV):
            acc0 = acc0 + g0[pl.ds(off + i * 16, 16)]
            acc1 = acc1 + g1[pl.ds(off + i * 16, 16)]
        t0 = g0[pl.ds(off + _NV * 16, 16)]
        t1 = g1[pl.ds(off + _NV * 16, 16)]
        acc0 = acc0 + jnp.where(tail_mask, t0, 0.0)
        acc1 = acc1 + jnp.where(tail_mask, t1, 0.0)
        outb[s, pl.ds(0, 16)] = acc0
        outb[s, pl.ds(16, 16)] = acc1
        return carry

    lax.fori_loop(0, _SPW, acc_sample, 0)
    pltpu.sync_copy(outb, out_ref.at[pl.ds(base, _SPW)])


def _sc_gather(x_flat, tw0, tw1):
    mesh = plsc.VectorSubcoreMesh(core_axis_name="c", subcore_axis_name="s")
    f = pl.kernel(
        _sc_body,
        out_type=jax.ShapeDtypeStruct((_B, 32), jnp.float32),
        mesh=mesh,
        scratch_types=[
            pltpu.VMEM((_IPW,), jnp.int32),
            pltpu.VMEM((_IPW + _GPAD,), jnp.float32),
            pltpu.VMEM((_IPW + _GPAD,), jnp.float32),
            pltpu.VMEM((_SPW, 32), jnp.float32),
            pltpu.SemaphoreType.DMA,
        ],
        compiler_params=pltpu.CompilerParams(use_tc_tiling_on_sc=False),
    )
    return f(x_flat, tw0, tw1)


def _fold_body(part_ref, b_ref, out_ref):
    j = lax.broadcasted_iota(jnp.int32, (32, _C), 0)
    k = lax.broadcasted_iota(jnp.int32, (32, _C), 1)
    sel = ((j // 16) == k).astype(jnp.float32)
    out_ref[...] = lax.dot_general(
        part_ref[...], sel, (((1,), (0,)), ((), ())),
        preferred_element_type=jnp.float32) + b_ref[...]


def _fold(partial, fc_b2d):
    return pl.pallas_call(
        _fold_body,
        out_shape=jax.ShapeDtypeStruct((_B, _C), jnp.float32),
    )(partial, fc_b2d)


def kernel(x, emb_table, fc_w, fc_b):
    tw0, tw1 = _mm(fc_w, emb_table.T)
    partial = _sc_gather(x.reshape(-1), tw0, tw1)
    return _fold(partial, fc_b.reshape(1, _C))


# speedup vs baseline: 4.5718x; 1.1922x over previous
import jax
import jax.numpy as jnp
from jax import lax
from jax.experimental import pallas as pl
from jax.experimental.pallas import tpu as pltpu
from jax.experimental.pallas import tpu_sc as plsc

_VOCAB = 1000000
_D = 64
_C = 2
_B = 4096
_H = 200

_NC = 2
_NS = 16
_NW = _NC * _NS
_SPW = _B // _NW
_IPW = _SPW * _H
_NCHUNK = _IPW // 128
_GPAD = 16
_NV = _H // 16
_TAILR = _H - _NV * 16


_W1 = 16384


def _mm_body(w_ref, tabt_ref, out0_ref, out1_ref):
    w = w_ref[...] * (1.0 / _H)
    res = lax.dot_general(
        w, tabt_ref[...], (((1,), (0,)), ((), ())),
        preferred_element_type=jnp.float32)
    out0_ref[...] = res[0, :]
    out1_ref[...] = res[1, :]


def _mm(fc_w, tabt):
    grid = (pl.cdiv(_VOCAB, _W1),)
    return pl.pallas_call(
        _mm_body,
        grid=grid,
        in_specs=[
            pl.BlockSpec((_C, _D), lambda i: (0, 0)),
            pl.BlockSpec((_D, _W1), lambda i: (0, i)),
        ],
        out_specs=[
            pl.BlockSpec((_W1,), lambda i: (i,)),
            pl.BlockSpec((_W1,), lambda i: (i,)),
        ],
        out_shape=[
            jax.ShapeDtypeStruct((_VOCAB,), jnp.float32),
            jax.ShapeDtypeStruct((_VOCAB,), jnp.float32),
        ],
    )(fc_w, tabt)


def _sc_body(x_ref, tw0_ref, tw1_ref, out_ref, idx_v, g0, g1, outb, sem):
    wid = lax.axis_index("s") * _NC + lax.axis_index("c")
    base = pl.multiple_of(wid * _SPW, _SPW)
    pltpu.sync_copy(x_ref.at[pl.ds(base * _H, _IPW)], idx_v)

    def fire(c, carry):
        off = pl.multiple_of(c * 128, 128)
        ids = idx_v.at[pl.ds(off, 128)]
        pltpu.async_copy(tw0_ref.at[ids], g0.at[pl.ds(off, 128)], sem)
        pltpu.async_copy(tw1_ref.at[ids], g1.at[pl.ds(off, 128)], sem)
        return carry

    lax.fori_loop(0, _NCHUNK, fire, 0)
    pltpu.make_async_copy(
        tw0_ref.at[pl.ds(0, _IPW)], g0.at[pl.ds(0, _IPW)], sem).wait()
    pltpu.make_async_copy(
        tw1_ref.at[pl.ds(0, _IPW)], g1.at[pl.ds(0, _IPW)], sem).wait()

    lanes = lax.broadcasted_iota(jnp.int32, (16,), 0)
    tail_mask = lanes < _TAILR

    def acc_sample(s, carry):
        off = pl.multiple_of(s * _H, 8)
        acc0 = jnp.zeros((16,), jnp.float32)
        acc1 = jnp.zeros((16,), jnp.float32)
        for i in range(_NV):
            acc0 = acc0 + g0[pl.ds(off + i * 16, 16)]
            acc1 = acc1 + g1[pl.ds(off + i * 16, 16)]
        t0 = g0[pl.ds(off + _NV * 16, 16)]
        t1 = g1[pl.ds(off + _NV * 16, 16)]
        acc0 = acc0 + jnp.where(tail_mask, t0, 0.0)
        acc1 = acc1 + jnp.where(tail_mask, t1, 0.0)
        outb[s, pl.ds(0, 16)] = acc0
        outb[s, pl.ds(16, 16)] = acc1
        return carry

    lax.fori_loop(0, _SPW, acc_sample, 0)
    pltpu.sync_copy(outb, out_ref.at[pl.ds(base, _SPW)])


def _sc_gather(x_flat, tw0, tw1):
    mesh = plsc.VectorSubcoreMesh(core_axis_name="c", subcore_axis_name="s")
    f = pl.kernel(
        _sc_body,
        out_type=jax.ShapeDtypeStruct((_B, 32), jnp.float32),
        mesh=mesh,
        scratch_types=[
            pltpu.VMEM((_IPW,), jnp.int32),
            pltpu.VMEM((_IPW + _GPAD,), jnp.float32),
            pltpu.VMEM((_IPW + _GPAD,), jnp.float32),
            pltpu.VMEM((_SPW, 32), jnp.float32),
            pltpu.SemaphoreType.DMA,
        ],
        compiler_params=pltpu.CompilerParams(use_tc_tiling_on_sc=False),
    )
    return f(x_flat, tw0, tw1)


def _fold_body(part_ref, b_ref, out_ref):
    j = lax.broadcasted_iota(jnp.int32, (32, _C), 0)
    k = lax.broadcasted_iota(jnp.int32, (32, _C), 1)
    sel = ((j // 16) == k).astype(jnp.float32)
    out_ref[...] = lax.dot_general(
        part_ref[...], sel, (((1,), (0,)), ((), ())),
        preferred_element_type=jnp.float32) + b_ref[...]


def _fold(partial, fc_b2d):
    return pl.pallas_call(
        _fold_body,
        out_shape=jax.ShapeDtypeStruct((_B, _C), jnp.float32),
    )(partial, fc_b2d)


def kernel(x, emb_table, fc_w, fc_b):
    tw0, tw1 = _mm(fc_w, emb_table.T)
    partial = _sc_gather(x.reshape(-1), tw0, tw1)
    return _fold(partial, fc_b.reshape(1, _C))


# speedup vs baseline: 5.4471x; 1.1915x over previous
import jax
import jax.numpy as jnp
from jax import lax
from jax.experimental import pallas as pl
from jax.experimental.pallas import tpu as pltpu
from jax.experimental.pallas import tpu_sc as plsc

_VOCAB = 1000000
_D = 64
_C = 2
_B = 4096
_H = 200

_NC = 2
_NS = 16
_NW = _NC * _NS
_SPW = _B // _NW
_IPW = _SPW * _H
_NCHUNK = _IPW // 128
_GPAD = 16
_NV = _H // 16
_TAILR = _H - _NV * 16


_W1 = 16384


def _mm_body(w_ref, tabt_ref, out_ref):
    w = w_ref[...] * (1.0 / _H)
    res = lax.dot_general(
        w, tabt_ref[...], (((1,), (0,)), ((), ())),
        preferred_element_type=jnp.float32)
    b0 = lax.bitcast_convert_type(res[0, :], jnp.uint32)
    b1 = lax.bitcast_convert_type(res[1, :], jnp.uint32)
    lo = (b0 + jnp.uint32(0x8000)) >> jnp.uint32(16)
    hi = (b1 + jnp.uint32(0x8000)) & jnp.uint32(0xFFFF0000)
    out_ref[...] = lo | hi


def _mm(fc_w, tabt):
    grid = (pl.cdiv(_VOCAB, _W1),)
    return pl.pallas_call(
        _mm_body,
        grid=grid,
        in_specs=[
            pl.BlockSpec((_C, _D), lambda i: (0, 0)),
            pl.BlockSpec((_D, _W1), lambda i: (0, i)),
        ],
        out_specs=pl.BlockSpec((_W1,), lambda i: (i,)),
        out_shape=jax.ShapeDtypeStruct((_VOCAB,), jnp.uint32),
    )(fc_w, tabt)


_NGRP = 8
_WPG = _IPW // _NGRP
_SPG = _SPW // _NGRP


def _sc_body(x_ref, twp_ref, out_ref, idx_v, g, outb, *sems):
    wid = lax.axis_index("s") * _NC + lax.axis_index("c")
    base = pl.multiple_of(wid * _SPW, _SPW)
    pltpu.sync_copy(x_ref.at[pl.ds(base * _H, _IPW)], idx_v)

    cpg = _NCHUNK // _NGRP
    for grp in range(_NGRP):
        sem_g = sems[grp]

        def fire(c, carry, _sem=sem_g, _cbase=grp * cpg):
            off = pl.multiple_of((_cbase + c) * 128, 128)
            pltpu.async_copy(
                twp_ref.at[idx_v.at[pl.ds(off, 128)]],
                g.at[pl.ds(off, 128)], _sem)
            return carry

        lax.fori_loop(0, cpg, fire, 0)

    lanes = lax.broadcasted_iota(jnp.int32, (16,), 0)
    tail_mask = lanes < _TAILR
    shift = jnp.uint32(16)
    himask = jnp.uint32(0xFFFF0000)

    def acc_sample(s, carry):
        off = pl.multiple_of(s * _H, 8)
        acc0 = jnp.zeros((16,), jnp.float32)
        acc1 = jnp.zeros((16,), jnp.float32)
        for i in range(_NV):
            w = g[pl.ds(off + i * 16, 16)]
            acc0 = acc0 + plsc.bitcast(w << shift, jnp.float32)
            acc1 = acc1 + plsc.bitcast(w & himask, jnp.float32)
        wt = g[pl.ds(off + _NV * 16, 16)]
        wt = jnp.where(tail_mask, wt, jnp.uint32(0))
        acc0 = acc0 + plsc.bitcast(wt << shift, jnp.float32)
        acc1 = acc1 + plsc.bitcast(wt & himask, jnp.float32)
        outb[s, pl.ds(0, 16)] = acc0
        outb[s, pl.ds(16, 16)] = acc1
        return carry

    for grp in range(_NGRP):
        pltpu.make_async_copy(
            twp_ref.at[pl.ds(0, _WPG)],
            g.at[pl.ds(grp * _WPG, _WPG)], sems[grp]).wait()
        lax.fori_loop(grp * _SPG, (grp + 1) * _SPG, acc_sample, 0)

    pltpu.sync_copy(outb, out_ref.at[pl.ds(base, _SPW)])


def _sc_gather(x_flat, twp):
    mesh = plsc.VectorSubcoreMesh(core_axis_name="c", subcore_axis_name="s")
    f = pl.kernel(
        _sc_body,
        out_type=jax.ShapeDtypeStruct((_B, 32), jnp.float32),
        mesh=mesh,
        scratch_types=[
            pltpu.VMEM((_IPW,), jnp.int32),
            pltpu.VMEM((_IPW + _GPAD,), jnp.uint32),
            pltpu.VMEM((_SPW, 32), jnp.float32),
        ] + [pltpu.SemaphoreType.DMA for _ in range(_NGRP)],
        compiler_params=pltpu.CompilerParams(
            use_tc_tiling_on_sc=False, needs_layout_passes=False),
    )
    return f(x_flat, twp)


def _fold_body(part_ref, b_ref, out_ref):
    j = lax.broadcasted_iota(jnp.int32, (32, _C), 0)
    k = lax.broadcasted_iota(jnp.int32, (32, _C), 1)
    sel = ((j // 16) == k).astype(jnp.float32)
    out_ref[...] = lax.dot_general(
        part_ref[...], sel, (((1,), (0,)), ((), ())),
        preferred_element_type=jnp.float32) + b_ref[...]


def _fold(partial, fc_b2d):
    return pl.pallas_call(
        _fold_body,
        out_shape=jax.ShapeDtypeStruct((_B, _C), jnp.float32),
    )(partial, fc_b2d)


def kernel(x, emb_table, fc_w, fc_b):
    twp = _mm(fc_w, emb_table.T)
    partial = _sc_gather(x.reshape(-1), twp)
    return _fold(partial, fc_b.reshape(1, _C))
